# baseline (speedup 1.0000x reference)
import functools

import jax
import jax.numpy as jnp
from jax import lax
from jax.experimental import pallas as pl
from jax.experimental.pallas import tpu as pltpu
from jax.experimental.pallas import tpu_sc as plsc

B, L = 4096, 20
V, EA, ED, P = 100000, 64, 128, 128
DESC = 768
N = B * L
EA2 = 2 * EA

NC, NS = 2, 16
NW = NC * NS
ROWS_W = N // NW
CH = 128
NCH = ROWS_W // CH
NCHP = 24
LANES = 16


def _sc_gather_body(table_hbm, idx_hbm, out_hbm, idx_v, idx2_v, rows_v, sem):
    wid = lax.axis_index("s") * NC + lax.axis_index("c")
    pltpu.sync_copy(idx_hbm.at[wid], idx_v)
    for r in range(NCH):
        for j in range(CH // LANES):
            sl = pl.ds(j * LANES, LANES)
            idx2_v[r, sl] = lax.shift_right_logical(idx_v[r, sl], 1)
    base = wid * ROWS_W
    for c in range(NCH):
        pltpu.async_copy(table_hbm.at[idx2_v.at[c]], rows_v, sem).wait()
        pltpu.sync_copy(rows_v, out_hbm.at[pl.ds(base + c * CH, CH)])


def _sc_gather(table, ids):
    gather = functools.partial(
        pl.kernel,
        out_type=jax.ShapeDtypeStruct((N, EA2), jnp.float32),
        mesh=plsc.VectorSubcoreMesh(core_axis_name="c", subcore_axis_name="s"),
        scratch_types=[
            pltpu.VMEM((NCHP, CH), jnp.int32),
            pltpu.VMEM((NCHP, CH), jnp.int32),
            pltpu.VMEM((CH, EA2), jnp.float32),
            pltpu.SemaphoreType.DMA,
        ],
        compiler_params=pltpu.CompilerParams(use_tc_tiling_on_sc=True),
    )(_sc_gather_body)
    return gather(table, ids)


def _prep_body(wp_ref, wf2_ref, wa_ref, bp_ref, bfc_ref, wc_ref, bc_ref, wst_ref):
    wc_ref[...] = jnp.dot(
        wp_ref[...], wf2_ref[...],
        preferred_element_type=jnp.float32,
        precision=lax.Precision.DEFAULT,
    )
    bc_ref[...] = jnp.dot(
        bp_ref[...], wf2_ref[...],
        preferred_element_type=jnp.float32,
        precision=lax.Precision.DEFAULT,
    ) + bfc_ref[...]
    wst_ref[...] = jnp.concatenate([wa_ref[...], wa_ref[...]], axis=0)


R = 2048


def _main_body(desc_ref, g_ref, ids_ref, an_ref, wc_ref, wst_ref, wl_ref, bc_ref, out_ref):
    acc = jnp.dot(
        desc_ref[...], wc_ref[...],
        preferred_element_type=jnp.float32,
        precision=lax.Precision.DEFAULT,
    )
    lane = lax.broadcasted_iota(jnp.int32, (R, EA2), 1)
    hi = (lane >= EA).astype(jnp.float32)
    par = jnp.mod(ids_ref[...], 2.0)
    emb_part = g_ref[...] * (1.0 - jnp.abs(hi - par))
    acc = acc + jnp.dot(
        emb_part, wst_ref[...],
        preferred_element_type=jnp.float32,
        precision=lax.Precision.DEFAULT,
    )
    acc = acc + an_ref[...] * wl_ref[...]
    acc = acc + bc_ref[...]
    out_ref[...] = acc


def kernel(action_name_ids, if_anchor, desc_vecs, emb_table, W_proj, b_proj, W_fc, b_fc):
    desc_t = desc_vecs.transpose(1, 0, 2).reshape(N, DESC)
    ids_t = action_name_ids.transpose(1, 0).reshape(N).astype(jnp.int32)
    anchor_t = if_anchor.transpose(1, 0).reshape(N, 1)

    table = emb_table.reshape(V // 2, EA2)

    ids = jnp.pad(
        ids_t.reshape(NW, NCH, CH),
        ((0, 0), (0, NCHP - NCH), (0, 0)),
    )

    wa = W_fc[:EA]
    wf2 = W_fc[EA:EA + ED]
    wl = W_fc[EA + ED:]

    wc, bc, wstack = pl.pallas_call(
        _prep_body,
        out_shape=[
            jax.ShapeDtypeStruct((DESC, P), jnp.float32),
            jax.ShapeDtypeStruct((1, P), jnp.float32),
            jax.ShapeDtypeStruct((EA2, P), jnp.float32),
        ],
    )(W_proj, wf2, wa, b_proj.reshape(1, ED), b_fc.reshape(1, P))

    g = _sc_gather(table, ids)

    out = pl.pallas_call(
        _main_body,
        grid=(N // R,),
        in_specs=[
            pl.BlockSpec((R, DESC), lambda i: (i, 0)),
            pl.BlockSpec((R, EA2), lambda i: (i, 0)),
            pl.BlockSpec((R, 1), lambda i: (i, 0)),
            pl.BlockSpec((R, 1), lambda i: (i, 0)),
            pl.BlockSpec((DESC, P), lambda i: (0, 0)),
            pl.BlockSpec((EA2, P), lambda i: (0, 0)),
            pl.BlockSpec((1, P), lambda i: (0, 0)),
            pl.BlockSpec((1, P), lambda i: (0, 0)),
        ],
        out_specs=pl.BlockSpec((R, P), lambda i: (i, 0)),
        out_shape=jax.ShapeDtypeStruct((N, P), jnp.float32),
        compiler_params=pltpu.CompilerParams(
            dimension_semantics=("arbitrary",),
        ),
    )(
        desc_t,
        g,
        ids_t.astype(jnp.float32).reshape(N, 1),
        anchor_t,
        wc,
        wstack,
        wl,
        bc,
    )
    return out.reshape(L, B, P).transpose(1, 0, 2)
